# Initial kernel scaffold; baseline (speedup 1.0000x reference)
import jax
import jax.numpy as jnp
from jax.experimental import pallas as pl


def kernel(x, table):
    raise NotImplementedError("write your pallas kernel here")


# speedup vs baseline: 5.0470x; 5.0470x over previous
import jax
import jax.numpy as jnp
from jax.experimental import pallas as pl


def _copy_body(t_ref, o_ref):
    o_ref[...] = jnp.broadcast_to(t_ref[...][None], o_ref.shape)


def kernel(x, table):
    B, S, D = x.shape
    CHUNK = 512
    out = pl.pallas_call(
        _copy_body,
        grid=(S // CHUNK,),
        in_specs=[pl.BlockSpec((CHUNK, D), lambda i: (i, 0))],
        out_specs=pl.BlockSpec((B, CHUNK, D), lambda i: (0, i, 0)),
        out_shape=jax.ShapeDtypeStruct((B, S, D), table.dtype),
    )(table)
    return out
